# Initial kernel scaffold; baseline (speedup 1.0000x reference)
import jax
import jax.numpy as jnp
from jax.experimental import pallas as pl


def kernel(nodes, edges, states, params, index1, index2, gnode, gbond):
    raise NotImplementedError("write your pallas kernel here")


# speedup vs baseline: 1.9018x; 1.9018x over previous
import functools

import jax
import jax.numpy as jnp
from jax import lax
from jax.experimental import pallas as pl
from jax.experimental.pallas import tpu as pltpu
from jax.experimental.pallas import tpu_sc as plsc

N = 10000
E = 160000
G = 64
D = 300
DP = 384
DH = 128
DU = 304
NPAD = 10240
EBR = 1000
NBR = 1000
EPS = 1e-5


def _softplus(x):
    return jnp.maximum(x, 0.0) + jnp.log1p(jnp.exp(-jnp.abs(x)))


def _norm_act(h, stats, gb, nrows):
    s = stats[0, :]
    ss = stats[1, :]
    mean = s / nrows
    var = ss / nrows - mean * mean
    inv = lax.rsqrt(var + EPS)
    g = gb[0, :]
    b = gb[1, :]
    return _softplus((h - mean[None, :]) * inv[None, :] * g[None, :] + b[None, :])


def _stats_update(stats_ref, h, step):
    @pl.when(step == 0)
    def _():
        stats_ref[...] = jnp.zeros_like(stats_ref)

    ps = jnp.sum(h, axis=0)
    pss = jnp.sum(h * h, axis=0)
    pad = jnp.zeros((6, h.shape[1]), jnp.float32)
    stats_ref[...] += jnp.concatenate([ps[None, :], pss[None, :], pad], axis=0)


def _rows(br):
    return pl.BlockSpec((br, None), lambda i: (i, 0))


def _full(shape):
    return pl.BlockSpec(shape, lambda i: tuple(0 for _ in shape))


def _acc_spec(shape):
    return pl.BlockSpec(shape, lambda i: tuple(0 for _ in shape))


def _rows_spec(br, ncols):
    return pl.BlockSpec((br, ncols), lambda i: (i, 0))


_GC = 40


def _sc_gather(table, idx3d_1, idx3d_2):
    nw = 32
    bpw = E // nw
    nch = bpw // _GC
    mesh = plsc.VectorSubcoreMesh(core_axis_name="c", subcore_axis_name="s")

    @functools.partial(
        pl.kernel,
        mesh=mesh,
        out_type=[
            jax.ShapeDtypeStruct((E, DP), jnp.float32),
            jax.ShapeDtypeStruct((E, DP), jnp.float32),
        ],
        scratch_types=[
            pltpu.VMEM((nch, _GC), jnp.int32),
            pltpu.VMEM((_GC, DP), jnp.float32),
            pltpu.SemaphoreType.DMA,
        ],
    )
    def k(tab_hbm, i1_hbm, i2_hbm, o1_hbm, o2_hbm, idx_v, rows_v, sem):
        wid = lax.axis_index("s") * 2 + lax.axis_index("c")

        def one_table(i_hbm, o_hbm):
            pltpu.sync_copy(i_hbm.at[wid], idx_v)

            def body(j, _):
                pltpu.async_copy(tab_hbm.at[idx_v.at[j]], rows_v, sem).wait()
                pltpu.sync_copy(rows_v, o_hbm.at[pl.ds(wid * bpw + j * _GC, _GC)])
                return 0

            lax.fori_loop(0, nch, body, 0)

        one_table(i1_hbm, o1_hbm)
        one_table(i2_hbm, o2_hbm)

    return k(table, idx3d_1, idx3d_2)


_SCC = 80


def _sc_scatter(et0, et1, et2, idx3d, zeros_init):
    nt = 16
    epw = E // nt
    nch = epw // _SCC
    npt = NPAD // nt
    mesh = plsc.VectorSubcoreMesh(core_axis_name="c", subcore_axis_name="s")

    @functools.partial(
        pl.kernel,
        mesh=mesh,
        out_type=[
            jax.ShapeDtypeStruct((NPAD, DH), jnp.float32),
            jax.ShapeDtypeStruct((NPAD, DH), jnp.float32),
            jax.ShapeDtypeStruct((NPAD, DH), jnp.float32),
        ],
        scratch_types=[
            pltpu.VMEM((nch, _SCC), jnp.int32),
            pltpu.VMEM((_SCC, DH), jnp.float32),
            pltpu.VMEM_SHARED((NPAD, DH), jnp.float32),
            pltpu.SemaphoreType.DMA,
        ],
    )
    def k(e0_hbm, e1_hbm, e2_hbm, idx_hbm, z_hbm, o0_hbm, o1_hbm, o2_hbm,
          idx_v, rows_v, acc, sem):
        cid = lax.axis_index("c")
        sid = lax.axis_index("s")
        r0 = sid * npt
        pltpu.sync_copy(idx_hbm.at[sid], idx_v)
        pltpu.sync_copy(z_hbm.at[pl.ds(r0, npt)], acc.at[pl.ds(r0, npt)])
        plsc.subcore_barrier()

        def scatter_all(src_hbm):
            def body(j, _):
                off = sid * epw + j * _SCC
                pltpu.sync_copy(src_hbm.at[pl.ds(off, _SCC)], rows_v)
                pltpu.sync_copy(rows_v, acc.at[idx_v.at[j]], add=True)
                return 0

            lax.fori_loop(0, nch, body, 0)

        @pl.when(cid == 0)
        def _():
            scatter_all(e0_hbm)

        @pl.when(cid == 1)
        def _():
            scatter_all(e1_hbm)

        plsc.subcore_barrier()

        @pl.when(cid == 0)
        def _():
            pltpu.sync_copy(acc.at[pl.ds(r0, npt)], o0_hbm.at[pl.ds(r0, npt)])

        @pl.when(cid == 1)
        def _():
            pltpu.sync_copy(acc.at[pl.ds(r0, npt)], o1_hbm.at[pl.ds(r0, npt)])

        plsc.subcore_barrier()

        @pl.when(cid == 0)
        def _():
            pltpu.sync_copy(z_hbm.at[pl.ds(r0, npt)], acc.at[pl.ds(r0, npt)])

        plsc.subcore_barrier()

        @pl.when(cid == 0)
        def _():
            scatter_all(e2_hbm)

        plsc.subcore_barrier()

        @pl.when(cid == 0)
        def _():
            pltpu.sync_copy(acc.at[pl.ds(r0, npt)], o2_hbm.at[pl.ds(r0, npt)])

    return k(et0, et1, et2, idx3d, zeros_init)


def _tc_states_proj(states, w):
    kdim = w.shape[1]

    def body(s_ref, w_ref, o_ref):
        o_ref[...] = jnp.dot(s_ref[...].astype(jnp.bfloat16), w_ref[...],
                             preferred_element_type=jnp.float32)

    return pl.pallas_call(
        body,
        out_shape=jax.ShapeDtypeStruct((G, kdim), jnp.float32),
    )(states, w)


def _tc_h1(fs, fr, edges, gbond2d, wa, wb, wc, sb):
    grid = (E // EBR,)

    def body(fs_ref, fr_ref, e_ref, gb_ref, wa_ref, wb_ref, wc_ref, sb_ref,
             h_ref, st_ref):
        i = pl.program_id(0)
        oh = (gb_ref[...] == lax.broadcasted_iota(jnp.int32, (EBR, G), 1))
        h = jnp.dot(fs_ref[...].astype(jnp.bfloat16), wa_ref[...],
                    preferred_element_type=jnp.float32)
        h += jnp.dot(fr_ref[...].astype(jnp.bfloat16), wb_ref[...],
                     preferred_element_type=jnp.float32)
        h += jnp.dot(e_ref[...].astype(jnp.bfloat16), wc_ref[...],
                     preferred_element_type=jnp.float32)
        h += jnp.dot(oh.astype(jnp.bfloat16), sb_ref[...],
                     preferred_element_type=jnp.float32)
        h_ref[...] = h
        _stats_update(st_ref, h, i)

    return pl.pallas_call(
        body,
        grid=grid,
        in_specs=[
            _rows_spec(EBR, DP), _rows_spec(EBR, DP), _rows_spec(EBR, D),
            _rows_spec(EBR, 1),
            _acc_spec((DP, 600)), _acc_spec((DP, 600)), _acc_spec((D, 600)),
            _acc_spec((G, 600)),
        ],
        out_specs=[_rows_spec(EBR, 600), _acc_spec((8, 600))],
        out_shape=[
            jax.ShapeDtypeStruct((E, 600), jnp.float32),
            jax.ShapeDtypeStruct((8, 600), jnp.float32),
        ],
    )(fs, fr, edges, gbond2d, wa, wb, wc, sb)


def _tc_mm(h_prev, stats, gb, w, nrows):
    rtot, kdim = h_prev.shape
    ndim = w.shape[1]
    br = EBR if rtot == E else NBR
    grid = (rtot // br,)

    def body(hp_ref, st_in_ref, gb_ref, w_ref, h_ref, st_ref):
        i = pl.program_id(0)
        x = _norm_act(hp_ref[...], st_in_ref[...], gb_ref[...], nrows)
        h = jnp.dot(x.astype(jnp.bfloat16), w_ref[...],
                    preferred_element_type=jnp.float32)
        h_ref[...] = h
        _stats_update(st_ref, h, i)

    return pl.pallas_call(
        body,
        grid=grid,
        in_specs=[
            _rows_spec(br, kdim), _acc_spec((8, kdim)), _acc_spec((8, kdim)),
            _acc_spec((kdim, ndim)),
        ],
        out_specs=[_rows_spec(br, ndim), _acc_spec((8, ndim))],
        out_shape=[
            jax.ShapeDtypeStruct((rtot, ndim), jnp.float32),
            jax.ShapeDtypeStruct((8, ndim), jnp.float32),
        ],
    )(h_prev, stats, gb, w)


def _tc_ekp_h4(h3, stats3, gb3, edges, gbond2d, v0):
    grid = (E // EBR,)

    def body(h3_ref, st3_ref, gb3_ref, e_ref, gbd_ref, v0_ref,
             ekp_ref, h4_ref, st4_ref, ue_ref):
        i = pl.program_id(0)
        ek = e_ref[...] + _norm_act(h3_ref[...], st3_ref[...], gb3_ref[...], E)
        ekp_ref[...] = ek
        ekb = ek.astype(jnp.bfloat16)
        h4 = jnp.dot(ekb, v0_ref[...], preferred_element_type=jnp.float32)
        h4_ref[...] = h4
        _stats_update(st4_ref, h4, i)
        oh = (gbd_ref[...] == lax.broadcasted_iota(jnp.int32, (EBR, G), 1))
        ekx = jnp.concatenate(
            [ekb, jnp.ones((EBR, 1), jnp.bfloat16),
             jnp.zeros((EBR, 3), jnp.bfloat16)], axis=1)

        @pl.when(i == 0)
        def _():
            ue_ref[...] = jnp.zeros_like(ue_ref)

        ue_ref[...] += lax.dot_general(
            oh.astype(jnp.bfloat16), ekx, (((0,), (0,)), ((), ())),
            preferred_element_type=jnp.float32)

    return pl.pallas_call(
        body,
        grid=grid,
        in_specs=[
            _rows_spec(EBR, D), _acc_spec((8, D)), _acc_spec((8, D)),
            _rows_spec(EBR, D), _rows_spec(EBR, 1), _acc_spec((D, 600)),
        ],
        out_specs=[_rows_spec(EBR, D), _rows_spec(EBR, 600),
                   _acc_spec((8, 600)), _acc_spec((G, DU))],
        out_shape=[
            jax.ShapeDtypeStruct((E, D), jnp.float32),
            jax.ShapeDtypeStruct((E, 600), jnp.float32),
            jax.ShapeDtypeStruct((8, 600), jnp.float32),
            jax.ShapeDtypeStruct((G, DU), jnp.float32),
        ],
    )(h3, stats3, gb3, edges, gbond2d, v0)


def _tc_et(h5, stats5, gb5):
    grid = (E // EBR,)

    def body(h5_ref, st_ref, gb_ref, s0_ref, s1_ref, s2_ref):
        et = _norm_act(h5_ref[...], st_ref[...], gb_ref[...], E)
        one = jnp.ones((EBR, 1), jnp.float32)
        zpad = jnp.zeros((EBR, 83), jnp.float32)
        s0_ref[...] = et[:, 0:128]
        s1_ref[...] = et[:, 128:256]
        s2_ref[...] = jnp.concatenate([et[:, 256:300], one, zpad], axis=1)

    return pl.pallas_call(
        body,
        grid=grid,
        in_specs=[_rows_spec(EBR, D), _acc_spec((8, D)), _acc_spec((8, D))],
        out_specs=[_rows_spec(EBR, DH), _rows_spec(EBR, DH),
                   _rows_spec(EBR, DH)],
        out_shape=[
            jax.ShapeDtypeStruct((E, DH), jnp.float32),
            jax.ShapeDtypeStruct((E, DH), jnp.float32),
            jax.ShapeDtypeStruct((E, DH), jnp.float32),
        ],
    )(h5, stats5, gb5)


def _tc_hv1(acc0, acc1, acc2, nodes, gnode2d, ua, ub, sn):
    grid = (N // NBR,)

    def body(a0_ref, a1_ref, a2_ref, n_ref, gn_ref, ua_ref, ub_ref, sn_ref,
             h_ref, st_ref):
        i = pl.program_id(0)
        cnt = jnp.maximum(a2_ref[:, 44], 1.0)
        inv = (1.0 / cnt)[:, None]
        agg = jnp.concatenate(
            [a0_ref[...], a1_ref[...], a2_ref[:, 0:44]], axis=1) * inv
        oh = (gn_ref[...] == lax.broadcasted_iota(jnp.int32, (NBR, G), 1))
        h = jnp.dot(agg.astype(jnp.bfloat16), ua_ref[...],
                    preferred_element_type=jnp.float32)
        h += jnp.dot(n_ref[...].astype(jnp.bfloat16), ub_ref[...],
                     preferred_element_type=jnp.float32)
        h += jnp.dot(oh.astype(jnp.bfloat16), sn_ref[...],
                     preferred_element_type=jnp.float32)
        h_ref[...] = h
        _stats_update(st_ref, h, i)

    return pl.pallas_call(
        body,
        grid=grid,
        in_specs=[
            _rows_spec(NBR, DH), _rows_spec(NBR, DH), _rows_spec(NBR, DH),
            _rows_spec(NBR, D), _rows_spec(NBR, 1),
            _acc_spec((D, 600)), _acc_spec((D, 600)), _acc_spec((G, 600)),
        ],
        out_specs=[_rows_spec(NBR, 600), _acc_spec((8, 600))],
        out_shape=[
            jax.ShapeDtypeStruct((N, 600), jnp.float32),
            jax.ShapeDtypeStruct((8, 600), jnp.float32),
        ],
    )(acc0, acc1, acc2, nodes, gnode2d, ua, ub, sn)


def _tc_vip(hv3, statsv3, gbv3, nodes, gnode2d):
    grid = (N // NBR,)

    def body(h_ref, st_ref, gb_ref, n_ref, gn_ref, v_ref, uv_ref):
        i = pl.program_id(0)
        vip = n_ref[...] + _norm_act(h_ref[...], st_ref[...], gb_ref[...], N)
        v_ref[...] = vip
        oh = (gn_ref[...] == lax.broadcasted_iota(jnp.int32, (NBR, G), 1))
        vx = jnp.concatenate(
            [vip.astype(jnp.bfloat16), jnp.ones((NBR, 1), jnp.bfloat16),
             jnp.zeros((NBR, 3), jnp.bfloat16)], axis=1)

        @pl.when(i == 0)
        def _():
            uv_ref[...] = jnp.zeros_like(uv_ref)

        uv_ref[...] += lax.dot_general(
            oh.astype(jnp.bfloat16), vx, (((0,), (0,)), ((), ())),
            preferred_element_type=jnp.float32)

    return pl.pallas_call(
        body,
        grid=grid,
        in_specs=[_rows_spec(NBR, D), _acc_spec((8, D)), _acc_spec((8, D)),
                  _rows_spec(NBR, D), _rows_spec(NBR, 1)],
        out_specs=[_rows_spec(NBR, D), _acc_spec((G, DU))],
        out_shape=[
            jax.ShapeDtypeStruct((N, D), jnp.float32),
            jax.ShapeDtypeStruct((G, DU), jnp.float32),
        ],
    )(hv3, statsv3, gbv3, nodes, gnode2d)


def _tc_u(ue_acc, uv_acc, states, w0, w1, w2, gb0, gb1, gb2):
    def bn_act(h, gb):
        mean = jnp.mean(h, axis=0)
        var = jnp.mean(h * h, axis=0) - mean * mean
        inv = lax.rsqrt(var + EPS)
        return _softplus((h - mean[None, :]) * inv[None, :] * gb[0:1, :]
                         + gb[1:2, :])

    def body(ue_ref, uv_ref, s_ref, w0_ref, w1_ref, w2_ref,
             g0_ref, g1_ref, g2_ref, o_ref):
        ue = ue_ref[:, :D] / jnp.maximum(ue_ref[:, D:D + 1], 1.0)
        uv = uv_ref[:, :D] / jnp.maximum(uv_ref[:, D:D + 1], 1.0)
        x = jnp.concatenate([ue, uv, s_ref[...]], axis=1)
        h = jnp.dot(x.astype(jnp.bfloat16), w0_ref[...],
                    preferred_element_type=jnp.float32)
        x = bn_act(h, g0_ref[...])
        h = jnp.dot(x.astype(jnp.bfloat16), w1_ref[...],
                    preferred_element_type=jnp.float32)
        x = bn_act(h, g1_ref[...])
        h = jnp.dot(x.astype(jnp.bfloat16), w2_ref[...],
                    preferred_element_type=jnp.float32)
        o_ref[...] = s_ref[...] + bn_act(h, g2_ref[...])

    return pl.pallas_call(
        body,
        out_shape=jax.ShapeDtypeStruct((G, D), jnp.float32),
    )(ue_acc, uv_acc, states, w0, w1, w2, gb0, gb1, gb2)


def _gb(p):
    return jnp.stack([p["gamma"], p["beta"]] + [jnp.zeros_like(p["gamma"])] * 6)


def kernel(nodes, edges, states, params, index1, index2, gnode, gbond):
    f32 = jnp.float32
    bf16 = jnp.bfloat16
    index1 = index1.astype(jnp.int32)
    index2 = index2.astype(jnp.int32)
    gnode = gnode.astype(jnp.int32)
    gbond = gbond.astype(jnp.int32)

    nodes_p = jnp.pad(nodes, ((0, 0), (0, DP - D)))
    i1_3d = index1.reshape(32, -1, _GC)
    i2_3d = index2.reshape(32, -1, _GC)
    i1_sc = index1.reshape(16, -1, _SCC)
    gbond2d = gbond.reshape(E, 1)
    gnode2d = gnode.reshape(N, 1)
    zinit = jnp.zeros((NPAD, DH), f32)

    pe = params["mlp_e"]
    pv = params["mlp_v"]
    pu = params["mlp_u"]
    pa = params["edge_agg"]
    w0 = pe[0]["W"]
    wa = jnp.pad(w0[0:300], ((0, DP - D), (0, 0))).astype(bf16)
    wb = jnp.pad(w0[300:600], ((0, DP - D), (0, 0))).astype(bf16)
    wc = w0[600:900].astype(bf16)
    wd = w0[900:1200]
    u0 = pv[0]["W"]
    ua = u0[0:300].astype(bf16)
    ub = u0[300:600].astype(bf16)
    uc = u0[600:900]

    fs, fr = _sc_gather(nodes_p, i1_3d, i2_3d)

    sbsn = _tc_states_proj(states, jnp.concatenate([wd, uc], axis=1).astype(bf16))
    sb = sbsn[:, :600].astype(bf16)
    sn = sbsn[:, 600:].astype(bf16)

    h1, st1 = _tc_h1(fs, fr, edges, gbond2d, wa, wb, wc, sb)
    h2, st2 = _tc_mm(h1, st1, _gb(pe[0]), pe[1]["W"].astype(bf16), E)
    h3, st3 = _tc_mm(h2, st2, _gb(pe[1]), pe[2]["W"].astype(bf16), E)

    e_k_p, h4, st4, ue_acc = _tc_ekp_h4(h3, st3, _gb(pe[2]), edges, gbond2d,
                                        pa[0]["W"].astype(bf16))
    h5, st5 = _tc_mm(h4, st4, _gb(pa[0]), pa[1]["W"].astype(bf16), E)
    et0, et1, et2 = _tc_et(h5, st5, _gb(pa[1]))

    acc0, acc1, acc2 = _sc_scatter(et0, et1, et2, i1_sc, zinit)

    hv1, sv1 = _tc_hv1(acc0, acc1, acc2, nodes, gnode2d, ua, ub, sn)
    hv2, sv2 = _tc_mm(hv1, sv1, _gb(pv[0]), pv[1]["W"].astype(bf16), N)
    hv3, sv3 = _tc_mm(hv2, sv2, _gb(pv[1]), pv[2]["W"].astype(bf16), N)
    v_i_p, uv_acc = _tc_vip(hv3, sv3, _gb(pv[2]), nodes, gnode2d)

    u_p = _tc_u(ue_acc, uv_acc, states,
                pu[0]["W"].astype(bf16), pu[1]["W"].astype(bf16),
                pu[2]["W"].astype(bf16), _gb(pu[0]), _gb(pu[1]), _gb(pu[2]))

    return (v_i_p, e_k_p, u_p)
